# Initial kernel scaffold; baseline (speedup 1.0000x reference)
import jax
import jax.numpy as jnp
from jax.experimental import pallas as pl


def kernel(sequence, states_1, states_2, table, W, U, b):
    raise NotImplementedError("write your pallas kernel here")


# speedup vs baseline: 3.3960x; 3.3960x over previous
import functools

import jax
import jax.numpy as jnp
from jax import lax
from jax.experimental import pallas as pl
from jax.experimental.pallas import tpu as pltpu
from jax.experimental.pallas import tpu_sc as plsc

B, T, V, D, H = 1024, 200, 100000, 128, 64
TB = T * B
CHUNK = 128


def _sc_gather(idx_flat, table):
    info = plsc.get_sparse_core_info()
    nw = info.num_cores * info.num_subcores
    per_w = TB // nw
    n_chunks = per_w // CHUNK
    mesh = plsc.VectorSubcoreMesh(core_axis_name="c", subcore_axis_name="s")

    @functools.partial(
        pl.kernel,
        mesh=mesh,
        out_type=jax.ShapeDtypeStruct((TB, D), jnp.float32),
        scratch_types=[
            pltpu.VMEM((CHUNK,), jnp.int32),
            pltpu.VMEM((CHUNK, D), jnp.float32),
            pltpu.SemaphoreType.DMA,
        ],
    )
    def gather_kernel(idx_hbm, table_hbm, out_hbm, idx_v, rows_v, sem):
        wid = lax.axis_index("s") * info.num_cores + lax.axis_index("c")
        base = wid * per_w

        def body(j, carry):
            r0 = base + j * CHUNK
            pltpu.sync_copy(idx_hbm.at[pl.ds(r0, CHUNK)], idx_v)
            pltpu.async_copy(table_hbm.at[idx_v], rows_v, sem).wait()
            pltpu.sync_copy(rows_v, out_hbm.at[pl.ds(r0, CHUNK)])
            return carry

        lax.fori_loop(0, n_chunks, body, 0)

    return gather_kernel(idx_flat, table)


def _lstm_body(emb_ref, h0_ref, c0_ref, w_ref, u_ref, b_ref,
               out_ref, ht_ref, ct_ref, h_s, c_s):
    t = pl.program_id(0)

    @pl.when(t == 0)
    def _():
        h_s[...] = h0_ref[...]
        c_s[...] = c0_ref[...]

    x = emb_ref[0]
    h = h_s[...]
    z = (jnp.dot(x, w_ref[...], preferred_element_type=jnp.float32)
         + jnp.dot(h, u_ref[...], preferred_element_type=jnp.float32)
         + b_ref[...])
    i = jax.nn.sigmoid(z[:, :H])
    f = jax.nn.sigmoid(z[:, H:2 * H])
    g = jnp.tanh(z[:, 2 * H:3 * H])
    o = jax.nn.sigmoid(z[:, 3 * H:])
    c = f * c_s[...] + i * g
    hn = o * jnp.tanh(c)
    h_s[...] = hn
    c_s[...] = c
    out_ref[0] = hn

    @pl.when(t == T - 1)
    def _():
        ht_ref[...] = hn
        ct_ref[...] = c


def _tc_lstm(embed_tbd, h0, c0, w, u, b2d):
    return pl.pallas_call(
        _lstm_body,
        grid=(T,),
        in_specs=[
            pl.BlockSpec((1, B, D), lambda t: (t, 0, 0)),
            pl.BlockSpec((B, H), lambda t: (0, 0)),
            pl.BlockSpec((B, H), lambda t: (0, 0)),
            pl.BlockSpec((D, 4 * H), lambda t: (0, 0)),
            pl.BlockSpec((H, 4 * H), lambda t: (0, 0)),
            pl.BlockSpec((1, 4 * H), lambda t: (0, 0)),
        ],
        out_specs=[
            pl.BlockSpec((1, B, H), lambda t: (t, 0, 0)),
            pl.BlockSpec((B, H), lambda t: (0, 0)),
            pl.BlockSpec((B, H), lambda t: (0, 0)),
        ],
        out_shape=[
            jax.ShapeDtypeStruct((T, B, H), jnp.float32),
            jax.ShapeDtypeStruct((B, H), jnp.float32),
            jax.ShapeDtypeStruct((B, H), jnp.float32),
        ],
        scratch_shapes=[
            pltpu.VMEM((B, H), jnp.float32),
            pltpu.VMEM((B, H), jnp.float32),
        ],
    )(embed_tbd, h0, c0, w, u, b2d)


def kernel(sequence, states_1, states_2, table, W, U, b):
    idx_flat = jnp.transpose(sequence).reshape(TB)
    embed = _sc_gather(idx_flat, table).reshape(T, B, D)
    out_tbh, h_t, c_t = _tc_lstm(embed, states_1, states_2, W, U,
                                 b.reshape(1, 4 * H))
    return jnp.swapaxes(out_tbh, 0, 1), h_t, c_t
